# baseline (speedup 1.0000x reference)
import functools

import jax
import jax.numpy as jnp
from jax import lax
from jax.experimental import pallas as pl
from jax.experimental.pallas import tpu as pltpu
from jax.experimental.pallas import tpu_sc as plsc

_N = 10000
_E = 320000
_D = 128
_DH = 256
_NC, _NS = 2, 16
_NT = _NC * _NS
_CH = 64
_EPT = 10240
_EP = _EPT * _NT
_NCHUNK = _EPT // _CH
_PCH = 64
_NB = 4
_TCH = _EP // _CH
_CA = 224
_CB = _TCH // _NS - _CA
_PHA = (64, 64, 64, 32)
_PHB = (64, 32)
_CPT = _EP // _CH
_NPAD = 10112
_RPT = _NPAD // _NS
_NH = 10240


@functools.cache
def _mesh():
    return plsc.VectorSubcoreMesh(core_axis_name="c", subcore_axis_name="s",
                                  num_cores=_NC, num_subcores=_NS)


@functools.cache
def _sc_hist_call():
    return pl.kernel(
        _sc_hist_body,
        out_type=jax.ShapeDtypeStruct((_NT, 2 * _NH), jnp.float32),
        mesh=_mesh(),
        compiler_params=pltpu.CompilerParams(needs_layout_passes=False),
        scratch_types=[
            pltpu.VMEM((_NCHUNK, _CH), jnp.int32),
            pltpu.VMEM((_NCHUNK, _CH), jnp.int32),
            pltpu.VMEM((2 * _NH,), jnp.float32),
        ],
    )


def _sc_hist_body(src_hbm, dst_hbm, out_hbm, sidx, didx, hist):
    cid = lax.axis_index("c")
    sid = lax.axis_index("s")
    tid = cid * _NS + sid

    def zero_body(r, carry):
        hist[pl.ds(r * 16, 16)] = jnp.zeros((16,), jnp.float32)
        return carry

    lax.fori_loop(0, (2 * _NH) // 16, zero_body, 0)

    pltpu.sync_copy(src_hbm.at[pl.ds(tid * _NCHUNK, _NCHUNK)], sidx)
    pltpu.sync_copy(dst_hbm.at[pl.ds(tid * _NCHUNK, _NCHUNK)], didx)

    ones16 = jnp.ones((16,), jnp.float32)

    def chunk_body(k, carry):
        for j in range(_CH // 16):
            iv_s = sidx[k, pl.ds(j * 16, 16)]
            plsc.addupdate_scatter(hist, [iv_s], ones16)
            iv_d = didx[k, pl.ds(j * 16, 16)] + _NH
            plsc.addupdate_scatter(hist, [iv_d], ones16)
        return carry

    lax.fori_loop(0, _NCHUNK, chunk_body, 0)

    pltpu.sync_copy(hist, out_hbm.at[tid])


@functools.cache
def _sc_agg_call():
    return pl.kernel(
        _sc_agg_body,
        out_type=jax.ShapeDtypeStruct((_NC, _NPAD, _D), jnp.float32),
        mesh=_mesh(),
        compiler_params=pltpu.CompilerParams(needs_layout_passes=False),
        scratch_types=[
            pltpu.VMEM((2, _PCH, _CH), jnp.int32),
            pltpu.VMEM((_NB, _CH, _D), jnp.float32),
            pltpu.VMEM_SHARED((_NPAD, _D), jnp.float32),
            [pltpu.SemaphoreType.DMA] * _NB,
            [pltpu.SemaphoreType.DMA] * _NB,
        ],
    )


def _sc_agg_body(h_hbm, src_hbm, dst_hbm, out_hbm,
                 idx, rows, acc, gsem, ssem):
    cid = lax.axis_index("c")
    sid = lax.axis_index("s")

    def zero_body(r, carry):
        for j in range(_D // 16):
            rows[0, r, pl.ds(j * 16, 16)] = jnp.zeros((16,), jnp.float32)
        return carry

    lax.fori_loop(0, _CH, zero_body, 0)
    row0 = sid * _RPT
    nfull = _RPT // _CH
    for k in range(nfull):
        pltpu.sync_copy(rows.at[0], acc.at[pl.ds(row0 + k * _CH, _CH)])
    rem = _RPT - nfull * _CH
    if rem:
        pltpu.sync_copy(rows.at[0, pl.ds(0, rem)],
                        acc.at[pl.ds(row0 + nfull * _CH, rem)])
    plsc.subcore_barrier()

    def start_gather(b, k):
        pltpu.async_copy(h_hbm.at[idx.at[0, k]], rows.at[b], gsem[b])

    def wait_gather(b):
        pltpu.make_async_copy(h_hbm.at[idx.at[0, 0]], rows.at[b],
                              gsem[b]).wait()

    def start_scatter(b, k):
        pltpu.async_copy(rows.at[b], acc.at[idx.at[1, k]], ssem[b], add=True)

    def wait_scatter(b):
        pltpu.make_async_copy(rows.at[b], acc.at[idx.at[1, 0]],
                              ssem[b]).wait()

    def run_phases(tile_base, phases):
        pdone = 0
        for pch in phases:
            pbase = tile_base + pdone
            pdone += pch
            pltpu.sync_copy(src_hbm.at[pl.ds(pbase, pch)],
                            idx.at[0, pl.ds(0, pch)])
            pltpu.sync_copy(dst_hbm.at[pl.ds(pbase, pch)],
                            idx.at[1, pl.ds(0, pch)])

            for b in range(_NB - 1):
                start_gather(b, b)
            for b in range(_NB):
                wait_gather(b)
                start_scatter(b, b)
                if b == 0:
                    start_gather(_NB - 1, _NB - 1)
                else:
                    wait_scatter(b - 1)
                    start_gather(b - 1, b + _NB - 1)

            def body(g, carry):
                k = _NB * g
                for b in range(_NB):
                    wait_gather(b)
                    start_scatter(b, k + b)
                    wait_scatter((b + _NB - 1) % _NB)
                    start_gather((b + _NB - 1) % _NB, k + b + _NB - 1)
                return carry

            lax.fori_loop(1, pch // _NB - 1, body, 0)

            kl = pch - _NB
            for b in range(_NB):
                wait_gather(b)
                start_scatter(b, kl + b)
                if b == 0:
                    wait_scatter(_NB - 1)
                    start_gather(_NB - 1, pch - 1)
                else:
                    wait_scatter(b - 1)
            wait_scatter(_NB - 1)

    @pl.when(cid == 0)
    def _():
        run_phases(sid * _CA, _PHA)

    @pl.when(cid == 1)
    def _():
        run_phases(_NS * _CA + sid * _CB, _PHB)

    plsc.subcore_barrier()
    pltpu.sync_copy(acc.at[pl.ds(row0, _RPT)],
                    out_hbm.at[cid, pl.ds(row0, _RPT)])


def _prep_body(hist_ref, feat_ref, h0_ref, rt_ref):
    deg = jnp.sum(hist_ref[...], axis=0)
    r2 = lax.rsqrt(jnp.maximum(deg.reshape(2, _NH), 1.0))
    rt = lax.dot_general(r2, jnp.eye(2, dtype=jnp.float32),
                         (((0,), (0,)), ((), ())),
                         preferred_element_type=jnp.float32)
    rt_ref[...] = rt
    h0_ref[...] = feat_ref[...] * rt[0:_N, 0:1]


def _tc_prep(hist, features):
    return pl.pallas_call(
        _prep_body,
        out_shape=[
            jax.ShapeDtypeStruct((_N, _D), jnp.float32),
            jax.ShapeDtypeStruct((_NH, 2), jnp.float32),
        ],
    )(hist, features)


_RB = 2000


def _mid_body(parts_ref, rt_ref, w1_ref, b1_ref, w2_ref, out_ref):
    agg = parts_ref[0] + parts_ref[1]
    r_out = rt_ref[:, 0:1]
    r_in = rt_ref[:, 1:2]
    t = jnp.dot(agg, w1_ref[...], preferred_element_type=jnp.float32)
    t = jnp.maximum(t * r_in + b1_ref[...][None, :], 0.0)
    out_ref[...] = jnp.dot(t * r_out, w2_ref[...],
                           preferred_element_type=jnp.float32)


def _tc_mid(parts, rt, w1, b1, w2):
    return pl.pallas_call(
        _mid_body,
        grid=(_N // _RB,),
        in_specs=[
            pl.BlockSpec((_NC, _RB, _D), lambda i: (0, i, 0)),
            pl.BlockSpec((_RB, 2), lambda i: (i, 0)),
            pl.BlockSpec((_D, _DH), lambda i: (0, 0)),
            pl.BlockSpec((_DH,), lambda i: (0,)),
            pl.BlockSpec((_DH, _D), lambda i: (0, 0)),
        ],
        out_specs=pl.BlockSpec((_RB, _D), lambda i: (i, 0)),
        out_shape=jax.ShapeDtypeStruct((_N, _D), jnp.float32),
    )(parts, rt, w1, b1, w2)


def _out_body(parts_ref, rt_ref, b2_ref, out_ref):
    agg = parts_ref[0] + parts_ref[1]
    out_ref[...] = jnp.maximum(agg * rt_ref[:, 1:2] + b2_ref[...][None, :],
                               0.0)


def _tc_out(parts, rt, b2):
    return pl.pallas_call(
        _out_body,
        grid=(_N // _RB,),
        in_specs=[
            pl.BlockSpec((_NC, _RB, _D), lambda i: (0, i, 0)),
            pl.BlockSpec((_RB, 2), lambda i: (i, 0)),
            pl.BlockSpec((_D,), lambda i: (0,)),
        ],
        out_specs=pl.BlockSpec((_RB, _D), lambda i: (i, 0)),
        out_shape=jax.ShapeDtypeStruct((_N, _D), jnp.float32),
    )(parts, rt, b2)


def kernel(features, edge_index, W1, b1, W2, b2):
    src = edge_index[0]
    dst = edge_index[1]
    pad = _EP - _E
    src_g = jnp.concatenate([src, jnp.zeros((pad,), jnp.int32)])
    src_h = jnp.concatenate([src, jnp.full((pad,), _NH - 1, jnp.int32)])
    dst_p = jnp.concatenate([dst, jnp.full((pad,), _N, jnp.int32)])
    src_g = src_g.reshape(_CPT, _CH)
    src_h = src_h.reshape(_CPT, _CH)
    dst_p = dst_p.reshape(_CPT, _CH)

    hist = _sc_hist_call()(src_h, dst_p)
    h0, rt = _tc_prep(hist, features)
    parts1 = _sc_agg_call()(h0, src_g, dst_p)
    p2 = _tc_mid(parts1, rt, W1, b1, W2)
    parts2 = _sc_agg_call()(p2, src_g, dst_p)
    return _tc_out(parts2, rt, b2)


# speedup vs baseline: 1.0345x; 1.0345x over previous
import functools

import jax
import jax.numpy as jnp
from jax import lax
from jax.experimental import pallas as pl
from jax.experimental.pallas import tpu as pltpu
from jax.experimental.pallas import tpu_sc as plsc

_N = 10000
_E = 320000
_D = 128
_DH = 256
_NC, _NS = 2, 16
_NT = _NC * _NS
_CH = 64
_EPT = 10240
_EP = _EPT * _NT
_NCHUNK = _EPT // _CH
_PCH = 64
_NB = 4
_TCH = _EP // _CH
_CA = 256
_CB = _TCH // _NS - _CA
_PHA = (64, 64, 64, 64)
_PHB = (64,)
_CPT = _EP // _CH
_NPAD = 10112
_RPT = _NPAD // _NS
_NH = 10240


@functools.cache
def _mesh():
    return plsc.VectorSubcoreMesh(core_axis_name="c", subcore_axis_name="s",
                                  num_cores=_NC, num_subcores=_NS)


@functools.cache
def _sc_hist_call():
    return pl.kernel(
        _sc_hist_body,
        out_type=jax.ShapeDtypeStruct((_NT, 2 * _NH), jnp.float32),
        mesh=_mesh(),
        compiler_params=pltpu.CompilerParams(needs_layout_passes=False),
        scratch_types=[
            pltpu.VMEM((_NCHUNK, _CH), jnp.int32),
            pltpu.VMEM((_NCHUNK, _CH), jnp.int32),
            pltpu.VMEM((2 * _NH,), jnp.float32),
        ],
    )


def _sc_hist_body(src_hbm, dst_hbm, out_hbm, sidx, didx, hist):
    cid = lax.axis_index("c")
    sid = lax.axis_index("s")
    tid = cid * _NS + sid

    def zero_body(r, carry):
        hist[pl.ds(r * 16, 16)] = jnp.zeros((16,), jnp.float32)
        return carry

    lax.fori_loop(0, (2 * _NH) // 16, zero_body, 0)

    pltpu.sync_copy(src_hbm.at[pl.ds(tid * _NCHUNK, _NCHUNK)], sidx)
    pltpu.sync_copy(dst_hbm.at[pl.ds(tid * _NCHUNK, _NCHUNK)], didx)

    ones16 = jnp.ones((16,), jnp.float32)

    def chunk_body(k, carry):
        for j in range(_CH // 16):
            iv_s = sidx[k, pl.ds(j * 16, 16)]
            plsc.addupdate_scatter(hist, [iv_s], ones16)
            iv_d = didx[k, pl.ds(j * 16, 16)] + _NH
            plsc.addupdate_scatter(hist, [iv_d], ones16)
        return carry

    lax.fori_loop(0, _NCHUNK, chunk_body, 0)

    pltpu.sync_copy(hist, out_hbm.at[tid])


@functools.cache
def _sc_agg_call():
    return pl.kernel(
        _sc_agg_body,
        out_type=jax.ShapeDtypeStruct((_NC, _NPAD, _D), jnp.float32),
        mesh=_mesh(),
        compiler_params=pltpu.CompilerParams(needs_layout_passes=False),
        scratch_types=[
            pltpu.VMEM((2, _PCH, _CH), jnp.int32),
            pltpu.VMEM((_NB, _CH, _D), jnp.float32),
            pltpu.VMEM_SHARED((_NPAD, _D), jnp.float32),
            [pltpu.SemaphoreType.DMA] * _NB,
            [pltpu.SemaphoreType.DMA] * _NB,
        ],
    )


def _sc_agg_body(h_hbm, src_hbm, dst_hbm, out_hbm,
                 idx, rows, acc, gsem, ssem):
    cid = lax.axis_index("c")
    sid = lax.axis_index("s")

    def zero_body(r, carry):
        for j in range(_D // 16):
            rows[0, r, pl.ds(j * 16, 16)] = jnp.zeros((16,), jnp.float32)
        return carry

    lax.fori_loop(0, _CH, zero_body, 0)
    row0 = sid * _RPT
    nfull = _RPT // _CH
    for k in range(nfull):
        pltpu.sync_copy(rows.at[0], acc.at[pl.ds(row0 + k * _CH, _CH)])
    rem = _RPT - nfull * _CH
    if rem:
        pltpu.sync_copy(rows.at[0, pl.ds(0, rem)],
                        acc.at[pl.ds(row0 + nfull * _CH, rem)])
    plsc.subcore_barrier()

    def start_gather(b, k):
        pltpu.async_copy(h_hbm.at[idx.at[0, k]], rows.at[b], gsem[b])

    def wait_gather(b):
        pltpu.make_async_copy(h_hbm.at[idx.at[0, 0]], rows.at[b],
                              gsem[b]).wait()

    def start_scatter(b, k):
        pltpu.async_copy(rows.at[b], acc.at[idx.at[1, k]], ssem[b], add=True)

    def wait_scatter(b):
        pltpu.make_async_copy(rows.at[b], acc.at[idx.at[1, 0]],
                              ssem[b]).wait()

    def run_phases(tile_base, phases):
        pdone = 0
        for pch in phases:
            pbase = tile_base + pdone
            pdone += pch
            pltpu.sync_copy(src_hbm.at[pl.ds(pbase, pch)],
                            idx.at[0, pl.ds(0, pch)])
            pltpu.sync_copy(dst_hbm.at[pl.ds(pbase, pch)],
                            idx.at[1, pl.ds(0, pch)])

            for b in range(_NB - 1):
                start_gather(b, b)
            for b in range(_NB):
                wait_gather(b)
                start_scatter(b, b)
                if b == 0:
                    start_gather(_NB - 1, _NB - 1)
                else:
                    wait_scatter(b - 1)
                    start_gather(b - 1, b + _NB - 1)

            def body(g, carry):
                k = _NB * g
                for b in range(_NB):
                    wait_gather(b)
                    start_scatter(b, k + b)
                    wait_scatter((b + _NB - 1) % _NB)
                    start_gather((b + _NB - 1) % _NB, k + b + _NB - 1)
                return carry

            lax.fori_loop(1, pch // _NB - 1, body, 0)

            kl = pch - _NB
            for b in range(_NB):
                wait_gather(b)
                start_scatter(b, kl + b)
                if b == 0:
                    wait_scatter(_NB - 1)
                    start_gather(_NB - 1, pch - 1)
                else:
                    wait_scatter(b - 1)
            wait_scatter(_NB - 1)

    @pl.when(cid == 0)
    def _():
        run_phases(sid * _CA, _PHA)

    @pl.when(cid == 1)
    def _():
        run_phases(_NS * _CA + sid * _CB, _PHB)

    plsc.subcore_barrier()
    pltpu.sync_copy(acc.at[pl.ds(row0, _RPT)],
                    out_hbm.at[cid, pl.ds(row0, _RPT)])


def _prep_body(hist_ref, feat_ref, h0_ref, rt_ref):
    deg = jnp.sum(hist_ref[...], axis=0)
    r2 = lax.rsqrt(jnp.maximum(deg.reshape(2, _NH), 1.0))
    rt = lax.dot_general(r2, jnp.eye(2, dtype=jnp.float32),
                         (((0,), (0,)), ((), ())),
                         preferred_element_type=jnp.float32)
    rt_ref[...] = rt
    h0_ref[...] = feat_ref[...] * rt[0:_N, 0:1]


def _tc_prep(hist, features):
    return pl.pallas_call(
        _prep_body,
        out_shape=[
            jax.ShapeDtypeStruct((_N, _D), jnp.float32),
            jax.ShapeDtypeStruct((_NH, 2), jnp.float32),
        ],
    )(hist, features)


_RB = 2000


def _mid_body(parts_ref, rt_ref, w1_ref, b1_ref, w2_ref, out_ref):
    agg = parts_ref[0] + parts_ref[1]
    r_out = rt_ref[:, 0:1]
    r_in = rt_ref[:, 1:2]
    t = jnp.dot(agg, w1_ref[...], preferred_element_type=jnp.float32)
    t = jnp.maximum(t * r_in + b1_ref[...][None, :], 0.0)
    out_ref[...] = jnp.dot(t * r_out, w2_ref[...],
                           preferred_element_type=jnp.float32)


def _tc_mid(parts, rt, w1, b1, w2):
    return pl.pallas_call(
        _mid_body,
        grid=(_N // _RB,),
        in_specs=[
            pl.BlockSpec((_NC, _RB, _D), lambda i: (0, i, 0)),
            pl.BlockSpec((_RB, 2), lambda i: (i, 0)),
            pl.BlockSpec((_D, _DH), lambda i: (0, 0)),
            pl.BlockSpec((_DH,), lambda i: (0,)),
            pl.BlockSpec((_DH, _D), lambda i: (0, 0)),
        ],
        out_specs=pl.BlockSpec((_RB, _D), lambda i: (i, 0)),
        out_shape=jax.ShapeDtypeStruct((_N, _D), jnp.float32),
    )(parts, rt, w1, b1, w2)


def _out_body(parts_ref, rt_ref, b2_ref, out_ref):
    agg = parts_ref[0] + parts_ref[1]
    out_ref[...] = jnp.maximum(agg * rt_ref[:, 1:2] + b2_ref[...][None, :],
                               0.0)


def _tc_out(parts, rt, b2):
    return pl.pallas_call(
        _out_body,
        grid=(_N // _RB,),
        in_specs=[
            pl.BlockSpec((_NC, _RB, _D), lambda i: (0, i, 0)),
            pl.BlockSpec((_RB, 2), lambda i: (i, 0)),
            pl.BlockSpec((_D,), lambda i: (0,)),
        ],
        out_specs=pl.BlockSpec((_RB, _D), lambda i: (i, 0)),
        out_shape=jax.ShapeDtypeStruct((_N, _D), jnp.float32),
    )(parts, rt, b2)


def kernel(features, edge_index, W1, b1, W2, b2):
    src = edge_index[0]
    dst = edge_index[1]
    pad = _EP - _E
    src_g = jnp.concatenate([src, jnp.zeros((pad,), jnp.int32)])
    src_h = jnp.concatenate([src, jnp.full((pad,), _NH - 1, jnp.int32)])
    dst_p = jnp.concatenate([dst, jnp.full((pad,), _N, jnp.int32)])
    src_g = src_g.reshape(_CPT, _CH)
    src_h = src_h.reshape(_CPT, _CH)
    dst_p = dst_p.reshape(_CPT, _CH)

    hist = _sc_hist_call()(src_h, dst_p)
    h0, rt = _tc_prep(hist, features)
    parts1 = _sc_agg_call()(h0, src_g, dst_p)
    p2 = _tc_mid(parts1, rt, W1, b1, W2)
    parts2 = _sc_agg_call()(p2, src_g, dst_p)
    return _tc_out(parts2, rt, b2)


# speedup vs baseline: 1.0411x; 1.0064x over previous
import functools

import jax
import jax.numpy as jnp
from jax import lax
from jax.experimental import pallas as pl
from jax.experimental.pallas import tpu as pltpu
from jax.experimental.pallas import tpu_sc as plsc

_N = 10000
_E = 320000
_D = 128
_DH = 256
_NC, _NS = 2, 16
_NT = _NC * _NS
_CH = 64
_EPT = 10240
_EP = _EPT * _NT
_NCHUNK = _EPT // _CH
_PCH = 64
_NB = 4
_TCH = _EP // _CH
_CA = 264
_CB = _TCH // _NS - _CA
_PHA = (64, 64, 64, 64, 8)
_PHB = (56,)
_CPT = _EP // _CH
_NPAD = 10112
_RPT = _NPAD // _NS
_NH = 10240


@functools.cache
def _mesh():
    return plsc.VectorSubcoreMesh(core_axis_name="c", subcore_axis_name="s",
                                  num_cores=_NC, num_subcores=_NS)


@functools.cache
def _sc_hist_call():
    return pl.kernel(
        _sc_hist_body,
        out_type=jax.ShapeDtypeStruct((_NT, 2 * _NH), jnp.float32),
        mesh=_mesh(),
        compiler_params=pltpu.CompilerParams(needs_layout_passes=False),
        scratch_types=[
            pltpu.VMEM((_NCHUNK, _CH), jnp.int32),
            pltpu.VMEM((_NCHUNK, _CH), jnp.int32),
            pltpu.VMEM((2 * _NH,), jnp.float32),
        ],
    )


def _sc_hist_body(src_hbm, dst_hbm, out_hbm, sidx, didx, hist):
    cid = lax.axis_index("c")
    sid = lax.axis_index("s")
    tid = cid * _NS + sid

    def zero_body(r, carry):
        hist[pl.ds(r * 16, 16)] = jnp.zeros((16,), jnp.float32)
        return carry

    lax.fori_loop(0, (2 * _NH) // 16, zero_body, 0)

    pltpu.sync_copy(src_hbm.at[pl.ds(tid * _NCHUNK, _NCHUNK)], sidx)
    pltpu.sync_copy(dst_hbm.at[pl.ds(tid * _NCHUNK, _NCHUNK)], didx)

    ones16 = jnp.ones((16,), jnp.float32)

    def chunk_body(k, carry):
        for j in range(_CH // 16):
            iv_s = sidx[k, pl.ds(j * 16, 16)]
            plsc.addupdate_scatter(hist, [iv_s], ones16)
            iv_d = didx[k, pl.ds(j * 16, 16)] + _NH
            plsc.addupdate_scatter(hist, [iv_d], ones16)
        return carry

    lax.fori_loop(0, _NCHUNK, chunk_body, 0)

    pltpu.sync_copy(hist, out_hbm.at[tid])


@functools.cache
def _sc_agg_call():
    return pl.kernel(
        _sc_agg_body,
        out_type=jax.ShapeDtypeStruct((_NC, _NPAD, _D), jnp.float32),
        mesh=_mesh(),
        compiler_params=pltpu.CompilerParams(needs_layout_passes=False),
        scratch_types=[
            pltpu.VMEM((2, _PCH, _CH), jnp.int32),
            pltpu.VMEM((_NB, _CH, _D), jnp.float32),
            pltpu.VMEM_SHARED((_NPAD, _D), jnp.float32),
            [pltpu.SemaphoreType.DMA] * _NB,
            [pltpu.SemaphoreType.DMA] * _NB,
        ],
    )


def _sc_agg_body(h_hbm, src_hbm, dst_hbm, out_hbm,
                 idx, rows, acc, gsem, ssem):
    cid = lax.axis_index("c")
    sid = lax.axis_index("s")

    def zero_body(r, carry):
        for j in range(_D // 16):
            rows[0, r, pl.ds(j * 16, 16)] = jnp.zeros((16,), jnp.float32)
        return carry

    lax.fori_loop(0, _CH, zero_body, 0)
    row0 = sid * _RPT
    nfull = _RPT // _CH
    rem = _RPT - nfull * _CH
    for k in range(nfull):
        pltpu.async_copy(rows.at[0], acc.at[pl.ds(row0 + k * _CH, _CH)],
                         gsem[0])
    if rem:
        pltpu.async_copy(rows.at[0, pl.ds(0, rem)],
                         acc.at[pl.ds(row0 + nfull * _CH, rem)], gsem[0])
    for k in range(nfull):
        pltpu.make_async_copy(rows.at[0], acc.at[pl.ds(row0 + k * _CH, _CH)],
                              gsem[0]).wait()
    if rem:
        pltpu.make_async_copy(rows.at[0, pl.ds(0, rem)],
                              acc.at[pl.ds(row0 + nfull * _CH, rem)],
                              gsem[0]).wait()
    plsc.subcore_barrier()

    def start_gather(b, k):
        pltpu.async_copy(h_hbm.at[idx.at[0, k]], rows.at[b], gsem[b])

    def wait_gather(b):
        pltpu.make_async_copy(h_hbm.at[idx.at[0, 0]], rows.at[b],
                              gsem[b]).wait()

    def start_scatter(b, k):
        pltpu.async_copy(rows.at[b], acc.at[idx.at[1, k]], ssem[b], add=True)

    def wait_scatter(b):
        pltpu.make_async_copy(rows.at[b], acc.at[idx.at[1, 0]],
                              ssem[b]).wait()

    def run_phases(tile_base, phases):
        pdone = 0
        for pch in phases:
            pbase = tile_base + pdone
            pdone += pch
            pltpu.async_copy(src_hbm.at[pl.ds(pbase, pch)],
                             idx.at[0, pl.ds(0, pch)], gsem[0])
            pltpu.async_copy(dst_hbm.at[pl.ds(pbase, pch)],
                             idx.at[1, pl.ds(0, pch)], gsem[1])
            pltpu.make_async_copy(src_hbm.at[pl.ds(pbase, pch)],
                                  idx.at[0, pl.ds(0, pch)], gsem[0]).wait()
            pltpu.make_async_copy(dst_hbm.at[pl.ds(pbase, pch)],
                                  idx.at[1, pl.ds(0, pch)], gsem[1]).wait()

            for b in range(_NB - 1):
                start_gather(b, b)
            for b in range(_NB):
                wait_gather(b)
                start_scatter(b, b)
                if b == 0:
                    start_gather(_NB - 1, _NB - 1)
                else:
                    wait_scatter(b - 1)
                    start_gather(b - 1, b + _NB - 1)

            def body(g, carry):
                k = _NB * g
                for b in range(_NB):
                    wait_gather(b)
                    start_scatter(b, k + b)
                    wait_scatter((b + _NB - 1) % _NB)
                    start_gather((b + _NB - 1) % _NB, k + b + _NB - 1)
                return carry

            lax.fori_loop(1, pch // _NB - 1, body, 0)

            kl = pch - _NB
            for b in range(_NB):
                wait_gather(b)
                start_scatter(b, kl + b)
                if b == 0:
                    wait_scatter(_NB - 1)
                    start_gather(_NB - 1, pch - 1)
                else:
                    wait_scatter(b - 1)
            wait_scatter(_NB - 1)

    @pl.when(cid == 0)
    def _():
        run_phases(sid * _CA, _PHA)

    @pl.when(cid == 1)
    def _():
        run_phases(_NS * _CA + sid * _CB, _PHB)

    plsc.subcore_barrier()
    pltpu.sync_copy(acc.at[pl.ds(row0, _RPT)],
                    out_hbm.at[cid, pl.ds(row0, _RPT)])


def _prep_body(hist_ref, feat_ref, h0_ref, rt_ref):
    deg = jnp.sum(hist_ref[...], axis=0)
    r2 = lax.rsqrt(jnp.maximum(deg.reshape(2, _NH), 1.0))
    rt = lax.dot_general(r2, jnp.eye(2, dtype=jnp.float32),
                         (((0,), (0,)), ((), ())),
                         preferred_element_type=jnp.float32)
    rt_ref[...] = rt
    h0_ref[...] = feat_ref[...] * rt[0:_N, 0:1]


def _tc_prep(hist, features):
    return pl.pallas_call(
        _prep_body,
        out_shape=[
            jax.ShapeDtypeStruct((_N, _D), jnp.float32),
            jax.ShapeDtypeStruct((_NH, 2), jnp.float32),
        ],
    )(hist, features)


_RB = 2000


def _mid_body(parts_ref, rt_ref, w1_ref, b1_ref, w2_ref, out_ref):
    agg = parts_ref[0] + parts_ref[1]
    r_out = rt_ref[:, 0:1]
    r_in = rt_ref[:, 1:2]
    t = jnp.dot(agg, w1_ref[...], preferred_element_type=jnp.float32)
    t = jnp.maximum(t * r_in + b1_ref[...][None, :], 0.0)
    out_ref[...] = jnp.dot(t * r_out, w2_ref[...],
                           preferred_element_type=jnp.float32)


def _tc_mid(parts, rt, w1, b1, w2):
    return pl.pallas_call(
        _mid_body,
        grid=(_N // _RB,),
        in_specs=[
            pl.BlockSpec((_NC, _RB, _D), lambda i: (0, i, 0)),
            pl.BlockSpec((_RB, 2), lambda i: (i, 0)),
            pl.BlockSpec((_D, _DH), lambda i: (0, 0)),
            pl.BlockSpec((_DH,), lambda i: (0,)),
            pl.BlockSpec((_DH, _D), lambda i: (0, 0)),
        ],
        out_specs=pl.BlockSpec((_RB, _D), lambda i: (i, 0)),
        out_shape=jax.ShapeDtypeStruct((_N, _D), jnp.float32),
    )(parts, rt, w1, b1, w2)


def _out_body(parts_ref, rt_ref, b2_ref, out_ref):
    agg = parts_ref[0] + parts_ref[1]
    out_ref[...] = jnp.maximum(agg * rt_ref[:, 1:2] + b2_ref[...][None, :],
                               0.0)


def _tc_out(parts, rt, b2):
    return pl.pallas_call(
        _out_body,
        grid=(_N // _RB,),
        in_specs=[
            pl.BlockSpec((_NC, _RB, _D), lambda i: (0, i, 0)),
            pl.BlockSpec((_RB, 2), lambda i: (i, 0)),
            pl.BlockSpec((_D,), lambda i: (0,)),
        ],
        out_specs=pl.BlockSpec((_RB, _D), lambda i: (i, 0)),
        out_shape=jax.ShapeDtypeStruct((_N, _D), jnp.float32),
    )(parts, rt, b2)


def kernel(features, edge_index, W1, b1, W2, b2):
    src = edge_index[0]
    dst = edge_index[1]
    pad = _EP - _E
    src_g = jnp.concatenate([src, jnp.zeros((pad,), jnp.int32)])
    src_h = jnp.concatenate([src, jnp.full((pad,), _NH - 1, jnp.int32)])
    dst_p = jnp.concatenate([dst, jnp.full((pad,), _N, jnp.int32)])
    src_g = src_g.reshape(_CPT, _CH)
    src_h = src_h.reshape(_CPT, _CH)
    dst_p = dst_p.reshape(_CPT, _CH)

    hist = _sc_hist_call()(src_h, dst_p)
    h0, rt = _tc_prep(hist, features)
    parts1 = _sc_agg_call()(h0, src_g, dst_p)
    p2 = _tc_mid(parts1, rt, W1, b1, W2)
    parts2 = _sc_agg_call()(p2, src_g, dst_p)
    return _tc_out(parts2, rt, b2)
